# baseline (speedup 1.0000x reference)
import functools

import jax
import jax.numpy as jnp
from jax.experimental import pallas as pl
from jax.experimental.pallas import tpu as pltpu

_BETA = 0.25


def _vq_body(nt, nk, tc, z_hbm, cb_ref, zq_hbm, idx_ref, loss_ref,
             zbuf, zqbuf, insem, outsem):
    nc = nt // tc
    cb = cb_ref[...]
    cbb = cb.astype(jnp.bfloat16)
    cbm2 = cb * -2.0
    cnorm = jnp.transpose(
        jnp.sum(cb * cb, axis=1, keepdims=True))

    kr = jax.lax.broadcasted_iota(jnp.int32, (1, nk), 1)
    arows = jnp.concatenate(
        [(kr // 32).astype(jnp.float32),
         (kr % 32).astype(jnp.float32)], axis=0).astype(jnp.bfloat16)
    liota = jax.lax.broadcasted_iota(jnp.int32, (tc, nk), 1)

    def in_copy(c):
        return pltpu.make_async_copy(
            z_hbm.at[pl.ds(c * tc, tc), :], zbuf.at[c % 2], insem.at[c % 2])

    def out_copy(c):
        return pltpu.make_async_copy(
            zqbuf.at[c % 2], zq_hbm.at[pl.ds(c * tc, tc), :], outsem.at[c % 2])

    in_copy(0).start()
    acc = jnp.zeros((1, 1), jnp.float32)
    for c in range(nc):
        if c + 1 < nc:
            in_copy(c + 1).start()
        in_copy(c).wait()
        z = zbuf[c % 2]

        s = jax.lax.dot_general(
            z, cbm2, (((1,), (1,)), ((), ())),
            preferred_element_type=jnp.float32)
        znorm = jnp.sum(z * z, axis=1, keepdims=True)
        dist = (znorm + s) + cnorm

        m = jnp.min(dist, axis=1, keepdims=True)
        idxc = jnp.min(jnp.where(dist == m, liota, nk),
                       axis=1, keepdims=True)
        onehot = (liota == idxc).astype(jnp.bfloat16)

        hilo = jax.lax.dot_general(
            arows, onehot, (((1,), (1,)), ((), ())),
            preferred_element_type=jnp.float32)
        idxrow = (32.0 * hilo[0:1] + hilo[1:2]).astype(jnp.int32)
        idx_ref[:, pl.ds(c * tc, tc)] = idxrow

        zq = jax.lax.dot_general(
            onehot, cbb, (((1,), (0,)), ((), ())),
            preferred_element_type=jnp.float32)
        d = zq - z
        if c >= 2:
            out_copy(c - 2).wait()
        zqbuf[c % 2] = z + d
        out_copy(c).start()
        acc = acc + jnp.sum(d * d, keepdims=True)

    mean = acc / (nt * cb_ref.shape[1])
    loss_ref[...] = _BETA * mean + mean
    out_copy(nc - 2).wait()
    out_copy(nc - 1).wait()


def kernel(z, codebook):
    B, C, H, W = z.shape
    K = codebook.shape[0]
    NT = B * H * W
    TC = 512
    zf = jnp.transpose(z, (0, 2, 3, 1)).reshape(NT, C)

    zqf, idx2, loss11 = pl.pallas_call(
        functools.partial(_vq_body, NT, K, TC),
        grid=(1,),
        in_specs=[
            pl.BlockSpec(memory_space=pl.ANY),
            pl.BlockSpec((K, C), lambda i: (0, 0)),
        ],
        out_specs=[
            pl.BlockSpec(memory_space=pl.ANY),
            pl.BlockSpec((1, NT), lambda i: (0, 0)),
            pl.BlockSpec((1, 1), lambda i: (0, 0)),
        ],
        out_shape=[
            jax.ShapeDtypeStruct((NT, C), jnp.float32),
            jax.ShapeDtypeStruct((1, NT), jnp.int32),
            jax.ShapeDtypeStruct((1, 1), jnp.float32),
        ],
        scratch_shapes=[
            pltpu.VMEM((2, TC, C), jnp.float32),
            pltpu.VMEM((2, TC, C), jnp.float32),
            pltpu.SemaphoreType.DMA((2,)),
            pltpu.SemaphoreType.DMA((2,)),
        ],
    )(zf, codebook)

    zq = jnp.transpose(zqf.reshape(B, H, W, C), (0, 3, 1, 2))
    idx = idx2.reshape(-1)
    loss = loss11.reshape(())
    return zq, idx, loss


# speedup vs baseline: 1.0395x; 1.0395x over previous
import functools

import jax
import jax.numpy as jnp
from jax.experimental import pallas as pl
from jax.experimental.pallas import tpu as pltpu

_BETA = 0.25


def _vq_body(nt, nk, tc, z_hbm, cb_ref, zq_hbm, idx_ref, loss_ref,
             zbuf, zqbuf, insem, outsem):
    nc = nt // tc
    cb = cb_ref[...]
    cbb = cb.astype(jnp.bfloat16)
    cbm2 = cb * -2.0
    cnorm = jnp.transpose(
        jnp.sum(cb * cb, axis=1, keepdims=True))

    kr = jax.lax.broadcasted_iota(jnp.int32, (1, nk), 1)
    arows = jnp.concatenate(
        [(kr // 32).astype(jnp.float32),
         (kr % 32).astype(jnp.float32)], axis=0).astype(jnp.bfloat16)
    liota = jax.lax.broadcasted_iota(jnp.int32, (tc, nk), 1)

    def in_copy(c):
        return pltpu.make_async_copy(
            z_hbm.at[pl.ds(c * tc, tc), :], zbuf.at[pl.ds(c * tc, tc), :],
            insem)

    def out_copy(c):
        return pltpu.make_async_copy(
            zqbuf.at[pl.ds(c * tc, tc), :], zq_hbm.at[pl.ds(c * tc, tc), :],
            outsem)

    for c in range(nc):
        in_copy(c).start()
    acc = jnp.zeros((1, 1), jnp.float32)
    for c in range(nc):
        in_copy(c).wait()
        z = zbuf[pl.ds(c * tc, tc), :]

        s = jax.lax.dot_general(
            z, cbm2, (((1,), (1,)), ((), ())),
            preferred_element_type=jnp.float32)
        znorm = jnp.sum(z * z, axis=1, keepdims=True)
        dist = (znorm + s) + cnorm

        m = jnp.min(dist, axis=1, keepdims=True)
        idxc = jnp.min(jnp.where(dist == m, liota, nk),
                       axis=1, keepdims=True)
        onehot = (liota == idxc).astype(jnp.bfloat16)

        hilo = jax.lax.dot_general(
            arows, onehot, (((1,), (1,)), ((), ())),
            preferred_element_type=jnp.float32)
        idxrow = (32.0 * hilo[0:1] + hilo[1:2]).astype(jnp.int32)
        idx_ref[:, pl.ds(c * tc, tc)] = idxrow

        zq = jax.lax.dot_general(
            onehot, cbb, (((1,), (0,)), ((), ())),
            preferred_element_type=jnp.float32)
        d = zq - z
        zqbuf[pl.ds(c * tc, tc), :] = z + d
        out_copy(c).start()
        acc = acc + jnp.sum(d * d, keepdims=True)

    mean = acc / (nt * cb_ref.shape[1])
    loss_ref[...] = _BETA * mean + mean
    for c in range(nc):
        out_copy(c).wait()


def kernel(z, codebook):
    B, C, H, W = z.shape
    K = codebook.shape[0]
    NT = B * H * W
    TC = 512
    zf = jnp.transpose(z, (0, 2, 3, 1)).reshape(NT, C)

    zqf, idx2, loss11 = pl.pallas_call(
        functools.partial(_vq_body, NT, K, TC),
        grid=(1,),
        in_specs=[
            pl.BlockSpec(memory_space=pl.ANY),
            pl.BlockSpec((K, C), lambda i: (0, 0)),
        ],
        out_specs=[
            pl.BlockSpec(memory_space=pl.ANY),
            pl.BlockSpec((1, NT), lambda i: (0, 0)),
            pl.BlockSpec((1, 1), lambda i: (0, 0)),
        ],
        out_shape=[
            jax.ShapeDtypeStruct((NT, C), jnp.float32),
            jax.ShapeDtypeStruct((1, NT), jnp.int32),
            jax.ShapeDtypeStruct((1, 1), jnp.float32),
        ],
        scratch_shapes=[
            pltpu.VMEM((NT, C), jnp.float32),
            pltpu.VMEM((NT, C), jnp.float32),
            pltpu.SemaphoreType.DMA,
            pltpu.SemaphoreType.DMA,
        ],
    )(zf, codebook)

    zq = jnp.transpose(zqf.reshape(B, H, W, C), (0, 3, 1, 2))
    idx = idx2.reshape(-1)
    loss = loss11.reshape(())
    return zq, idx, loss


# speedup vs baseline: 1.3864x; 1.3337x over previous
import functools

import jax
import jax.numpy as jnp
from jax.experimental import pallas as pl

_BETA = 0.25


def _vq_body(nt, nk, tc, z_ref, cb_ref, zq_ref, idx_ref, loss_ref):
    cb = cb_ref[...]
    cbb = cb.astype(jnp.bfloat16)
    cbm2 = cb * -2.0
    cnorm = jnp.transpose(
        jnp.sum(cb * cb, axis=1, keepdims=True))

    kr = jax.lax.broadcasted_iota(jnp.int32, (1, nk), 1)
    arows = jnp.concatenate(
        [(kr // 32).astype(jnp.float32),
         (kr % 32).astype(jnp.float32)], axis=0).astype(jnp.bfloat16)
    liota = jax.lax.broadcasted_iota(jnp.int32, (tc, nk), 1)

    acc = jnp.zeros((1, 1), jnp.float32)
    for c in range(nt // tc):
        t0 = c * tc
        z = z_ref[pl.ds(t0, tc), :]
        s = jax.lax.dot_general(
            z, cbm2, (((1,), (1,)), ((), ())),
            preferred_element_type=jnp.float32)
        znorm = jnp.sum(z * z, axis=1, keepdims=True)
        dist = (znorm + s) + cnorm

        idxc = jnp.argmin(dist, axis=1)[:, None]
        onehot = (liota == idxc).astype(jnp.bfloat16)

        hilo = jax.lax.dot_general(
            arows, onehot, (((1,), (1,)), ((), ())),
            preferred_element_type=jnp.float32)
        idxrow = (32.0 * hilo[0:1] + hilo[1:2]).astype(jnp.int32)
        idx_ref[:, pl.ds(t0, tc)] = idxrow

        zq = jax.lax.dot_general(
            onehot, cbb, (((1,), (0,)), ((), ())),
            preferred_element_type=jnp.float32)
        d = zq - z
        zq_ref[pl.ds(t0, tc), :] = z + d
        acc = acc + jnp.sum(d * d, keepdims=True)

    mean = acc / (nt * z_ref.shape[1])
    loss_ref[...] = _BETA * mean + mean


def kernel(z, codebook):
    B, C, H, W = z.shape
    K = codebook.shape[0]
    NT = B * H * W
    TC = 1536
    zf = jnp.transpose(z, (0, 2, 3, 1)).reshape(NT, C)

    zqf, idx2, loss11 = pl.pallas_call(
        functools.partial(_vq_body, NT, K, TC),
        grid=(1,),
        in_specs=[
            pl.BlockSpec((NT, C), lambda i: (0, 0)),
            pl.BlockSpec((K, C), lambda i: (0, 0)),
        ],
        out_specs=[
            pl.BlockSpec((NT, C), lambda i: (0, 0)),
            pl.BlockSpec((1, NT), lambda i: (0, 0)),
            pl.BlockSpec((1, 1), lambda i: (0, 0)),
        ],
        out_shape=[
            jax.ShapeDtypeStruct((NT, C), jnp.float32),
            jax.ShapeDtypeStruct((1, NT), jnp.int32),
            jax.ShapeDtypeStruct((1, 1), jnp.float32),
        ],
    )(zf, codebook)

    zq = jnp.transpose(zqf.reshape(B, H, W, C), (0, 3, 1, 2))
    idx = idx2.reshape(-1)
    loss = loss11.reshape(())
    return zq, idx, loss
